# baseline (speedup 1.0000x reference)
import jax
import jax.numpy as jnp
from jax import lax
from jax.experimental import pallas as pl
from jax.experimental.pallas import tpu as pltpu
from jax.experimental.pallas import tpu_sc as plsc

_B = 16384
_K = 32
_PACK = 128 // _K
_NW = 32
_BPW = _B // _NW
_CHUNK = 128
_NCHUNK = _BPW // _CHUNK

_mesh = plsc.VectorSubcoreMesh(core_axis_name="c", subcore_axis_name="s")


def _mf_body(uid_hbm, iid_hbm, uf_hbm, if_hbm, bf_hbm, out_hbm,
             uid_v, iid_v, u_rows, v_rows, b_flat, out_v, sem, bsem):
    wid = lax.axis_index("s") * 2 + lax.axis_index("c")
    base = wid * _BPW

    for c in range(_NCHUNK):
        pltpu.sync_copy(uid_hbm.at[pl.ds(base + c * _CHUNK, _CHUNK)],
                        uid_v.at[c])
        pltpu.sync_copy(iid_hbm.at[pl.ds(base + c * _CHUNK, _CHUNK)],
                        iid_v.at[c])

    bias_copies = [
        pltpu.async_copy(bf_hbm.at[iid_v.at[c]],
                         b_flat.at[pl.ds(c * _CHUNK, _CHUNK)], bsem)
        for c in range(_NCHUNK)
    ]
    for cp in bias_copies:
        cp.wait()

    iota16 = lax.iota(jnp.int32, 16)

    def chunk(c, carry):
        def fire(j, carry2):
            uvec = uid_v[c, pl.ds(j * 16, 16)]
            ivec = iid_v[c, pl.ds(j * 16, 16)]
            for l in range(16):
                r = j * 16 + l
                pltpu.async_copy(uf_hbm.at[uvec[l] >> 2], u_rows.at[r], sem)
                pltpu.async_copy(if_hbm.at[ivec[l] >> 2], v_rows.at[r], sem)
            return carry2

        lax.fori_loop(0, _CHUNK // 16, fire, 0)
        pltpu.make_async_copy(uf_hbm.at[pl.ds(0, _CHUNK)], u_rows, sem).wait()
        pltpu.make_async_copy(if_hbm.at[pl.ds(0, _CHUNK)], v_rows, sem).wait()

        def group(g, carry3):
            rows = iota16 + g * 16
            uoff = (uid_v[c, pl.ds(g * 16, 16)] & (_PACK - 1)) << 5
            ioff = (iid_v[c, pl.ds(g * 16, 16)] & (_PACK - 1)) << 5
            acc0 = b_flat[pl.ds(c * _CHUNK + g * 16, 16)]
            acc1 = jnp.zeros((16,), jnp.float32)
            acc2 = jnp.zeros((16,), jnp.float32)
            acc3 = jnp.zeros((16,), jnp.float32)
            accs = [acc0, acc1, acc2, acc3]
            for k in range(_K):
                u = plsc.load_gather(u_rows, [rows, uoff + k])
                v = plsc.load_gather(v_rows, [rows, ioff + k])
                accs[k % 4] = accs[k % 4] + u * v
            out_v[pl.ds(c * _CHUNK + g * 16, 16)] = (
                (accs[0] + accs[1]) + (accs[2] + accs[3]))
            return carry3

        lax.fori_loop(0, _CHUNK // 16, group, 0)
        return carry

    lax.fori_loop(0, _NCHUNK, chunk, 0)

    pltpu.sync_copy(out_v, out_hbm.at[pl.ds(base, _BPW)])


_mf_kernel = pl.kernel(
    _mf_body,
    out_type=jax.ShapeDtypeStruct((_B,), jnp.float32),
    mesh=_mesh,
    scratch_types=[
        pltpu.VMEM((_NCHUNK, _CHUNK), jnp.int32),
        pltpu.VMEM((_NCHUNK, _CHUNK), jnp.int32),
        pltpu.VMEM((_CHUNK, 128), jnp.float32),
        pltpu.VMEM((_CHUNK, 128), jnp.float32),
        pltpu.VMEM((_BPW,), jnp.float32),
        pltpu.VMEM((_BPW,), jnp.float32),
        pltpu.SemaphoreType.DMA,
        pltpu.SemaphoreType.DMA,
    ],
    compiler_params=pltpu.CompilerParams(needs_layout_passes=False),
)


def kernel(user_ids, item_ids, user_factors, item_factors, bias_factors):
    uids = user_ids.astype(jnp.int32)
    iids = item_ids.astype(jnp.int32)
    ufp = user_factors.reshape(-1, 128)
    ifp = item_factors.reshape(-1, 128)
    bias = bias_factors.reshape(-1)
    return _mf_kernel(uids, iids, ufp, ifp, bias)


# speedup vs baseline: 1.3929x; 1.3929x over previous
import jax
import jax.numpy as jnp
from jax import lax
from jax.experimental import pallas as pl
from jax.experimental.pallas import tpu as pltpu
from jax.experimental.pallas import tpu_sc as plsc

_B = 16384
_K = 32
_NW = 32
_BPW = _B // _NW
_CHUNK = 128
_NCHUNK = _BPW // _CHUNK

_mesh = plsc.VectorSubcoreMesh(core_axis_name="c", subcore_axis_name="s")


def _mf_body(uid_hbm, iid_hbm, uf_hbm, if_hbm, bf_hbm, out_hbm,
             uid_v, iid_v, u_rows, v_rows, b_flat, out_v, sem, bsem):
    wid = lax.axis_index("s") * 2 + lax.axis_index("c")
    base = wid * _BPW

    for c in range(_NCHUNK):
        pltpu.sync_copy(uid_hbm.at[pl.ds(base + c * _CHUNK, _CHUNK)],
                        uid_v.at[c])
        pltpu.sync_copy(iid_hbm.at[pl.ds(base + c * _CHUNK, _CHUNK)],
                        iid_v.at[c])

    bias_copies = [
        pltpu.async_copy(bf_hbm.at[iid_v.at[c]],
                         b_flat.at[pl.ds(c * _CHUNK, _CHUNK)], bsem)
        for c in range(_NCHUNK)
    ]
    for cp in bias_copies:
        cp.wait()

    iota16 = lax.iota(jnp.int32, 16)

    def chunk(c, carry):
        def fire(j, carry2):
            uvec = uid_v[c, pl.ds(j * 16, 16)]
            ivec = iid_v[c, pl.ds(j * 16, 16)]
            for l in range(16):
                r = j * 16 + l
                pltpu.async_copy(uf_hbm.at[uvec[l]], u_rows.at[r], sem)
                pltpu.async_copy(if_hbm.at[ivec[l]], v_rows.at[r], sem)
            return carry2

        lax.fori_loop(0, _CHUNK // 16, fire, 0)
        pltpu.make_async_copy(uf_hbm.at[pl.ds(0, _CHUNK)], u_rows, sem).wait()
        pltpu.make_async_copy(if_hbm.at[pl.ds(0, _CHUNK)], v_rows, sem).wait()

        def group(g, carry3):
            rows = iota16 + g * 16
            acc0 = b_flat[pl.ds(c * _CHUNK + g * 16, 16)]
            acc1 = jnp.zeros((16,), jnp.float32)
            acc2 = jnp.zeros((16,), jnp.float32)
            acc3 = jnp.zeros((16,), jnp.float32)
            accs = [acc0, acc1, acc2, acc3]
            for k in range(_K):
                kk = jnp.full((16,), k, jnp.int32)
                u = plsc.load_gather(u_rows, [rows, kk])
                v = plsc.load_gather(v_rows, [rows, kk])
                accs[k % 4] = accs[k % 4] + u * v
            out_v[pl.ds(c * _CHUNK + g * 16, 16)] = (
                (accs[0] + accs[1]) + (accs[2] + accs[3]))
            return carry3

        lax.fori_loop(0, _CHUNK // 16, group, 0)
        return carry

    lax.fori_loop(0, _NCHUNK, chunk, 0)

    pltpu.sync_copy(out_v, out_hbm.at[pl.ds(base, _BPW)])


_mf_kernel = pl.kernel(
    _mf_body,
    out_type=jax.ShapeDtypeStruct((_B,), jnp.float32),
    mesh=_mesh,
    scratch_types=[
        pltpu.VMEM((_NCHUNK, _CHUNK), jnp.int32),
        pltpu.VMEM((_NCHUNK, _CHUNK), jnp.int32),
        pltpu.VMEM((_CHUNK, _K), jnp.float32),
        pltpu.VMEM((_CHUNK, _K), jnp.float32),
        pltpu.VMEM((_BPW,), jnp.float32),
        pltpu.VMEM((_BPW,), jnp.float32),
        pltpu.SemaphoreType.DMA,
        pltpu.SemaphoreType.DMA,
    ],
    compiler_params=pltpu.CompilerParams(needs_layout_passes=False),
)


def kernel(user_ids, item_ids, user_factors, item_factors, bias_factors):
    uids = user_ids.astype(jnp.int32)
    iids = item_ids.astype(jnp.int32)
    bias = bias_factors @ jnp.ones((1,), jnp.float32)
    return _mf_kernel(uids, iids, user_factors, item_factors, bias)
